# baseline (speedup 1.0000x reference)
import functools

import jax
import jax.numpy as jnp
from jax import lax
from jax.experimental import pallas as pl
from jax.experimental.pallas import tpu as pltpu
from jax.experimental.pallas import tpu_sc as plsc

B, F, V, D = 4096, 26, 100000, 32
NC, NS = 2, 16
NW = NC * NS
BPW = B // NW
RPW = BPW * F
CH = 104
NCH = RPW // CH
LANES = 16
DH = D // LANES
NR = 4
SPR = BPW // NR
LPR = SPR * F
CPR = LPR // CH
GPR = SPR // LANES
NG = BPW // LANES
WROW = 4 * D


def _fm_body(idx_t, idx_g, offs4, emb4, lin, bias16, out,
             idxt_v, idxg_v, off_v, rows_v, lin_v, out_v, bias_v,
             emb_sem, lin_sem):
    wid = lax.axis_index("s") * NC + lax.axis_index("c")

    pltpu.sync_copy(idx_t.at[wid], idxt_v)
    pltpu.sync_copy(idx_g.at[wid], idxg_v)
    pltpu.sync_copy(offs4.at[wid], off_v)
    pltpu.sync_copy(bias16, bias_v)

    for j in range(NCH):
        pltpu.async_copy(lin.at[idxt_v.at[j]],
                         lin_v.at[pl.ds(j * CH, CH)], lin_sem)
    pltpu.make_async_copy(lin.at[pl.ds(0, RPW)], lin_v, lin_sem).wait()

    lane26 = lax.iota(jnp.int32, LANES) * F

    def round_body(r, carry):
        for k in range(CPR):
            pltpu.async_copy(emb4.at[idxg_v.at[r * CPR + k]],
                             rows_v.at[pl.ds(k * CH, CH), :], emb_sem)
        pltpu.make_async_copy(emb4.at[pl.ds(0, LPR), :], rows_v,
                              emb_sem).wait()

        def group_body(c, carry2):
            g = r * GPR + c
            offv = [off_v[g * F + f, :] for f in range(F)]
            ovec = jnp.zeros((LANES,), jnp.float32)
            for j in range(LANES):
                lb0 = (c * LANES + j) * F
                acc = [jnp.zeros((LANES,), jnp.float32) for _ in range(DH)]
                ssq = [jnp.zeros((LANES,), jnp.float32) for _ in range(DH)]
                for f in range(F):
                    off = pl.multiple_of(offv[f][j], D)
                    for h in range(DH):
                        v = rows_v[lb0 + f, pl.ds(off + h * LANES, LANES)]
                        acc[h] = acc[h] + v
                        ssq[h] = ssq[h] + v * v
                cross = acc[0] * acc[0] - ssq[0]
                for h in range(1, DH):
                    cross = cross + acc[h] * acc[h] - ssq[h]
                inter = 0.5 * jnp.sum(cross)
                ovec = jnp.where(lane26 == j * F, inter, ovec)

            lbase = lane26 + g * (LANES * F)
            lacc = bias_v[...]
            for f in range(F):
                lacc = lacc + plsc.load_gather(lin_v, [lbase + f])
            out_v[g, :] = ovec + lacc
            return carry2

        lax.fori_loop(0, GPR, group_body, 0)
        return carry

    lax.fori_loop(0, NR, round_body, 0)

    pltpu.sync_copy(out_v, out.at[wid])


_fm_sc = functools.partial(
    pl.kernel,
    out_type=jax.ShapeDtypeStruct((NW, BPW // LANES, LANES), jnp.float32),
    mesh=plsc.VectorSubcoreMesh(core_axis_name="c", subcore_axis_name="s",
                                num_cores=NC, num_subcores=NS),
    compiler_params=pltpu.CompilerParams(needs_layout_passes=False,
                                         use_tc_tiling_on_sc=False),
    scratch_types=[
        pltpu.VMEM((NCH, CH), jnp.int32),
        pltpu.VMEM((NCH, CH), jnp.int32),
        pltpu.VMEM((NG * F, LANES), jnp.int32),
        pltpu.VMEM((LPR, WROW), jnp.float32),
        pltpu.VMEM((RPW,), jnp.float32),
        pltpu.VMEM((BPW // LANES, LANES), jnp.float32),
        pltpu.VMEM((LANES,), jnp.float32),
        pltpu.SemaphoreType.DMA,
        pltpu.SemaphoreType.DMA,
    ],
)(_fm_body)


def kernel(indices, embed_tables, lin_tables, bias):
    offs = jnp.arange(F, dtype=jnp.int32) * V
    flat = indices + offs[None, :]
    idx_t = flat.reshape(NW, NCH, CH)
    idx_g = (flat >> 2).reshape(NW, NCH, CH)
    offs4 = ((flat & 3) * D).reshape(NW, NG, LANES, F)
    offs4 = offs4.transpose(0, 1, 3, 2).reshape(NW, NG * F, LANES)
    emb4 = embed_tables.reshape(F * V // 4, WROW)
    lin2 = lin_tables.reshape(F * V)
    bias16 = jnp.broadcast_to(bias.astype(jnp.float32), (LANES,))
    return _fm_sc(idx_t, idx_g, offs4, emb4, lin2, bias16).reshape(B)


# speedup vs baseline: 1.9309x; 1.9309x over previous
import functools

import jax
import jax.numpy as jnp
from jax import lax
from jax.experimental import pallas as pl
from jax.experimental.pallas import tpu as pltpu
from jax.experimental.pallas import tpu_sc as plsc

B, F, V, D = 4096, 26, 100000, 32
NC, NS = 2, 16
NW = NC * NS
BPW = B // NW
LANES = 16
NG = BPW // LANES


def _fm_body(idxT, emb2, lin, bias16, out,
             idx_v, rowsT_v, lin_v, out_v, bias_v, emb_sem, lin_sem):
    wid = lax.axis_index("s") * NC + lax.axis_index("c")
    base = wid * BPW

    pltpu.sync_copy(idxT.at[:, pl.ds(base, BPW)], idx_v)
    pltpu.sync_copy(bias16, bias_v)

    for f in range(F):
        pltpu.async_copy(lin.at[f].at[idx_v.at[f]], lin_v.at[f], lin_sem)

    def fire_emb(t, carry):
        f = lax.shift_right_logical(t, 5)
        pltpu.async_copy(emb2.at[t].at[idx_v.at[f]], rowsT_v.at[t], emb_sem)
        return carry

    lax.fori_loop(0, F * D, fire_emb, 0)

    pltpu.make_async_copy(lin.at[:, pl.ds(0, BPW)], lin_v, lin_sem).wait()
    pltpu.make_async_copy(emb2.at[:, pl.ds(0, BPW)], rowsT_v, emb_sem).wait()

    def group_body(c, carry):
        col = c * LANES

        def dim_body(d, inter):
            s = jnp.zeros((LANES,), jnp.float32)
            q = jnp.zeros((LANES,), jnp.float32)
            for f in range(F):
                v = rowsT_v[f * D + d, pl.ds(col, LANES)]
                s = s + v
                q = q + v * v
            return inter + s * s - q

        inter = lax.fori_loop(0, D, dim_body,
                              jnp.zeros((LANES,), jnp.float32))
        lacc = bias_v[...]
        for f in range(F):
            lacc = lacc + lin_v[f, pl.ds(col, LANES)]
        out_v[0, pl.ds(col, LANES)] = 0.5 * inter + lacc
        return carry

    lax.fori_loop(0, NG, group_body, 0)

    pltpu.sync_copy(out_v, out.at[wid])


_fm_sc = functools.partial(
    pl.kernel,
    out_type=jax.ShapeDtypeStruct((NW, 1, BPW), jnp.float32),
    mesh=plsc.VectorSubcoreMesh(core_axis_name="c", subcore_axis_name="s",
                                num_cores=NC, num_subcores=NS),
    compiler_params=pltpu.CompilerParams(needs_layout_passes=False,
                                         use_tc_tiling_on_sc=False),
    scratch_types=[
        pltpu.VMEM((F, BPW), jnp.int32),
        pltpu.VMEM((F * D, BPW), jnp.float32),
        pltpu.VMEM((F, BPW), jnp.float32),
        pltpu.VMEM((1, BPW), jnp.float32),
        pltpu.VMEM((LANES,), jnp.float32),
        pltpu.SemaphoreType.DMA,
        pltpu.SemaphoreType.DMA,
    ],
)(_fm_body)


def kernel(indices, embed_tables, lin_tables, bias):
    idxT = indices.T
    emb2 = embed_tables.transpose(0, 2, 1).reshape(F * D, V)
    bias16 = jnp.broadcast_to(bias.astype(jnp.float32), (LANES,))
    return _fm_sc(idxT, emb2, lin_tables, bias16).reshape(B)
